# Initial kernel scaffold; baseline (speedup 1.0000x reference)
import jax
import jax.numpy as jnp
from jax.experimental import pallas as pl


def kernel(x, edge_index, Wm0, bm0, Wn0, bn0, g0, be0, Wm1, bm1, Wn1, bn1, g1, be1):
    raise NotImplementedError("write your pallas kernel here")


# speedup vs baseline: 6.9349x; 6.9349x over previous
import functools

import jax
import jax.numpy as jnp
from jax import lax
from jax.experimental import pallas as pl
from jax.experimental.pallas import tpu as pltpu
from jax.experimental.pallas import tpu_sc as plsc

N = 10000
E = 320000
D = 128
EPS = 1e-3

NC = 2
NS = 16
NW = NC * NS
E_W = E // NW
CHUNK = 80
NCHUNK = E_W // CHUNK
N_PAD = 10240
ROWS_TILE = N_PAD // NS

BM = 1000


def _sc_segment_sum(P, src_w, dst_w):
  mesh = plsc.VectorSubcoreMesh(core_axis_name="c", subcore_axis_name="s")

  @functools.partial(
      pl.kernel,
      mesh=mesh,
      out_type=jax.ShapeDtypeStruct((NC, N_PAD, D), jnp.float32),
      scratch_types=[
          pltpu.VMEM((NCHUNK, CHUNK), jnp.int32),
          pltpu.VMEM((NCHUNK, CHUNK), jnp.int32),
          pltpu.VMEM((CHUNK, D), jnp.float32),
          pltpu.VMEM_SHARED((N_PAD, D), jnp.float32),
          pltpu.SemaphoreType.DMA,
      ],
  )
  def seg_sum(p_hbm, src_hbm, dst_hbm, out_hbm, src_v, dst_v, rows_v,
              agg_sh, sem):
    cid = lax.axis_index("c")
    sid = lax.axis_index("s")
    wid = sid * NC + cid

    pltpu.sync_copy(src_hbm.at[wid], src_v)
    pltpu.sync_copy(dst_hbm.at[wid], dst_v)

    def _zrow(i, _):
      r = i // (D // 16)
      c0 = (i % (D // 16)) * 16
      rows_v[r, pl.ds(c0, 16)] = jnp.zeros((16,), jnp.float32)
      return 0
    lax.fori_loop(0, CHUNK * (D // 16), _zrow, 0)

    def _zcp(j, _):
      pltpu.sync_copy(rows_v, agg_sh.at[pl.ds(sid * ROWS_TILE + j * CHUNK, CHUNK)])
      return 0
    lax.fori_loop(0, ROWS_TILE // CHUNK, _zcp, 0)
    plsc.subcore_barrier()

    def _edge_chunk(c, _):
      pltpu.async_copy(p_hbm.at[src_v.at[c]], rows_v, sem).wait()
      pltpu.sync_copy(rows_v, agg_sh.at[dst_v.at[c]], add=True)
      return 0
    lax.fori_loop(0, NCHUNK, _edge_chunk, 0)
    plsc.subcore_barrier()

    pltpu.sync_copy(
        agg_sh.at[pl.ds(sid * ROWS_TILE, ROWS_TILE)],
        out_hbm.at[cid, pl.ds(sid * ROWS_TILE, ROWS_TILE)])

  return seg_sum(P, src_w, dst_w)


def _tc_msg(h, Wm, bm):
  def body(h_ref, w_ref, b_ref, o_ref):
    o_ref[...] = jnp.maximum(
        jnp.dot(h_ref[...], w_ref[...], preferred_element_type=jnp.float32)
        + b_ref[...], 0.0)

  return pl.pallas_call(
      body,
      grid=(N // BM,),
      in_specs=[
          pl.BlockSpec((BM, D), lambda i: (i, 0)),
          pl.BlockSpec((D, D), lambda i: (0, 0)),
          pl.BlockSpec((1, D), lambda i: (0, 0)),
      ],
      out_specs=pl.BlockSpec((BM, D), lambda i: (i, 0)),
      out_shape=jax.ShapeDtypeStruct((N, D), jnp.float32),
  )(h, Wm, bm.reshape(1, D))


def _tc_update(h, agg2, WnT, WnB, bn, g, be):
  def body(h_ref, a0_ref, a1_ref, wt_ref, wb_ref, b_ref, g_ref, be_ref,
           o_ref, ctx_ref):
    hv = h_ref[...]
    agg = a0_ref[...] + a1_ref[...]
    t = (jnp.dot(hv, wt_ref[...], preferred_element_type=jnp.float32)
         + jnp.dot(agg, wb_ref[...], preferred_element_type=jnp.float32)
         + b_ref[...])
    hn = jnp.maximum(hv + jnp.maximum(t, 0.0), 0.0)
    mu = jnp.mean(hn, axis=1, keepdims=True)
    var = jnp.mean((hn - mu) ** 2, axis=1, keepdims=True)
    y = (hn - mu) * lax.rsqrt(var + EPS) * g_ref[...] + be_ref[...]
    o_ref[...] = y

    @pl.when(pl.program_id(0) == 0)
    def _init():
      ctx_ref[...] = jnp.zeros_like(ctx_ref)

    ctx_ref[...] += jnp.sum(y, axis=0, keepdims=True)

    @pl.when(pl.program_id(0) == N // BM - 1)
    def _fin():
      ctx_ref[...] = ctx_ref[...] * (1.0 / N)

  return pl.pallas_call(
      body,
      grid=(N // BM,),
      in_specs=[
          pl.BlockSpec((BM, D), lambda i: (i, 0)),
          pl.BlockSpec((BM, D), lambda i: (i, 0)),
          pl.BlockSpec((BM, D), lambda i: (i, 0)),
          pl.BlockSpec((D, D), lambda i: (0, 0)),
          pl.BlockSpec((D, D), lambda i: (0, 0)),
          pl.BlockSpec((1, D), lambda i: (0, 0)),
          pl.BlockSpec((1, D), lambda i: (0, 0)),
          pl.BlockSpec((1, D), lambda i: (0, 0)),
      ],
      out_specs=[
          pl.BlockSpec((BM, D), lambda i: (i, 0)),
          pl.BlockSpec((1, D), lambda i: (0, 0)),
      ],
      out_shape=[
          jax.ShapeDtypeStruct((N, D), jnp.float32),
          jax.ShapeDtypeStruct((1, D), jnp.float32),
      ],
  )(h, agg2[0], agg2[1], WnT, WnB, bn.reshape(1, D), g.reshape(1, D),
    be.reshape(1, D))


def kernel(x, edge_index, Wm0, bm0, Wn0, bn0, g0, be0, Wm1, bm1, Wn1, bn1,
           g1, be1):
  src_w = edge_index[0].reshape(NW, NCHUNK, CHUNK)
  dst_w = edge_index[1].reshape(NW, NCHUNK, CHUNK)

  def hop(h, Wm, bm, Wn, bn, g, be):
    P = _tc_msg(h, Wm, bm)
    agg2 = _sc_segment_sum(P, src_w, dst_w)[:, :N, :]
    return _tc_update(h, agg2, Wn[:D], Wn[D:], bn, g, be)

  h1, _ = hop(x, Wm0, bm0, Wn0, bn0, g0, be0)
  h2, ctx = hop(h1, Wm1, bm1, Wn1, bn1, g1, be1)
  return (h2, ctx)


# speedup vs baseline: 10.0898x; 1.4549x over previous
import functools

import jax
import jax.numpy as jnp
from jax import lax
from jax.experimental import pallas as pl
from jax.experimental.pallas import tpu as pltpu
from jax.experimental.pallas import tpu_sc as plsc

N = 10000
E = 320000
D = 128
EPS = 1e-3

NC = 2
NS = 16
NW = NC * NS
E_W = E // NW
CHUNK = 80
NCHUNK = E_W // CHUNK
NSEC = 5
SEC = NCHUNK // NSEC
SPAIR = SEC // 2
N_PAD = 10240
ROWS_TILE = N_PAD // NS

BM = 1000


def _sc_segment_sum(P, src_w, dst_w):
  mesh = plsc.VectorSubcoreMesh(core_axis_name="c", subcore_axis_name="s")

  @functools.partial(
      pl.kernel,
      mesh=mesh,
      out_type=jax.ShapeDtypeStruct((NC, N_PAD, D), jnp.float32),
      scratch_types=[
          pltpu.VMEM((SEC, CHUNK), jnp.int32),
          pltpu.VMEM((SEC, CHUNK), jnp.int32),
          pltpu.VMEM((CHUNK, D), jnp.float32),
          pltpu.VMEM((CHUNK, D), jnp.float32),
          pltpu.VMEM_SHARED((N_PAD, D), jnp.float32),
          pltpu.SemaphoreType.DMA,
          pltpu.SemaphoreType.DMA,
      ],
  )
  def seg_sum(p_hbm, src_hbm, dst_hbm, out_hbm, src_v, dst_v, rows0, rows1,
              agg_sh, gsem0, gsem1):
    cid = lax.axis_index("c")
    sid = lax.axis_index("s")
    wid = sid * NC + cid

    def _zrow(i, _):
      r = i // (D // 16)
      c0 = (i % (D // 16)) * 16
      rows0[r, pl.ds(c0, 16)] = jnp.zeros((16,), jnp.float32)
      return 0
    lax.fori_loop(0, CHUNK * (D // 16), _zrow, 0)

    def _zcp(j, _):
      pltpu.sync_copy(rows0, agg_sh.at[pl.ds(sid * ROWS_TILE + j * CHUNK, CHUNK)])
      return 0
    lax.fori_loop(0, ROWS_TILE // CHUNK, _zcp, 0)
    plsc.subcore_barrier()

    for sec in range(NSEC):
      pltpu.sync_copy(src_hbm.at[wid, sec], src_v)
      pltpu.sync_copy(dst_hbm.at[wid, sec], dst_v)
      pltpu.async_copy(p_hbm.at[src_v.at[0]], rows0, gsem0)
      pltpu.async_copy(p_hbm.at[src_v.at[1]], rows1, gsem1)

      def _pair(t, _):
        c0 = 2 * t
        pltpu.make_async_copy(p_hbm.at[src_v.at[c0]], rows0, gsem0).wait()
        pltpu.sync_copy(rows0, agg_sh.at[dst_v.at[c0]], add=True)

        @pl.when(c0 + 2 < SEC)
        def _g0():
          pltpu.async_copy(p_hbm.at[src_v.at[c0 + 2]], rows0, gsem0)

        c1 = c0 + 1
        pltpu.make_async_copy(p_hbm.at[src_v.at[c1]], rows1, gsem1).wait()
        pltpu.sync_copy(rows1, agg_sh.at[dst_v.at[c1]], add=True)

        @pl.when(c1 + 2 < SEC)
        def _g1():
          pltpu.async_copy(p_hbm.at[src_v.at[c1 + 2]], rows1, gsem1)
        return 0

      lax.fori_loop(0, SPAIR, _pair, 0)
      if SEC % 2:
        pltpu.make_async_copy(p_hbm.at[src_v.at[SEC - 1]], rows0, gsem0).wait()
        pltpu.sync_copy(rows0, agg_sh.at[dst_v.at[SEC - 1]], add=True)
    plsc.subcore_barrier()

    pltpu.sync_copy(
        agg_sh.at[pl.ds(sid * ROWS_TILE, ROWS_TILE)],
        out_hbm.at[cid, pl.ds(sid * ROWS_TILE, ROWS_TILE)])

  return seg_sum(P, src_w, dst_w)


def _tc_msg(h, Wm, bm):
  def body(h_ref, w_ref, b_ref, o_ref):
    o_ref[...] = jnp.maximum(
        jnp.dot(h_ref[...], w_ref[...], preferred_element_type=jnp.float32)
        + b_ref[...], 0.0)

  return pl.pallas_call(
      body,
      grid=(N // BM,),
      in_specs=[
          pl.BlockSpec((BM, D), lambda i: (i, 0)),
          pl.BlockSpec((D, D), lambda i: (0, 0)),
          pl.BlockSpec((1, D), lambda i: (0, 0)),
      ],
      out_specs=pl.BlockSpec((BM, D), lambda i: (i, 0)),
      out_shape=jax.ShapeDtypeStruct((N, D), jnp.float32),
  )(h, Wm, bm.reshape(1, D))


def _tc_update(h, agg2, WnT, WnB, bn, g, be):
  def body(h_ref, a0_ref, a1_ref, wt_ref, wb_ref, b_ref, g_ref, be_ref,
           o_ref, ctx_ref):
    hv = h_ref[...]
    agg = a0_ref[...] + a1_ref[...]
    t = (jnp.dot(hv, wt_ref[...], preferred_element_type=jnp.float32)
         + jnp.dot(agg, wb_ref[...], preferred_element_type=jnp.float32)
         + b_ref[...])
    hn = jnp.maximum(hv + jnp.maximum(t, 0.0), 0.0)
    mu = jnp.mean(hn, axis=1, keepdims=True)
    var = jnp.mean((hn - mu) ** 2, axis=1, keepdims=True)
    y = (hn - mu) * lax.rsqrt(var + EPS) * g_ref[...] + be_ref[...]
    o_ref[...] = y

    @pl.when(pl.program_id(0) == 0)
    def _init():
      ctx_ref[...] = jnp.zeros_like(ctx_ref)

    ctx_ref[...] += jnp.sum(y, axis=0, keepdims=True)

    @pl.when(pl.program_id(0) == N // BM - 1)
    def _fin():
      ctx_ref[...] = ctx_ref[...] * (1.0 / N)

  return pl.pallas_call(
      body,
      grid=(N // BM,),
      in_specs=[
          pl.BlockSpec((BM, D), lambda i: (i, 0)),
          pl.BlockSpec((BM, D), lambda i: (i, 0)),
          pl.BlockSpec((BM, D), lambda i: (i, 0)),
          pl.BlockSpec((D, D), lambda i: (0, 0)),
          pl.BlockSpec((D, D), lambda i: (0, 0)),
          pl.BlockSpec((1, D), lambda i: (0, 0)),
          pl.BlockSpec((1, D), lambda i: (0, 0)),
          pl.BlockSpec((1, D), lambda i: (0, 0)),
      ],
      out_specs=[
          pl.BlockSpec((BM, D), lambda i: (i, 0)),
          pl.BlockSpec((1, D), lambda i: (0, 0)),
      ],
      out_shape=[
          jax.ShapeDtypeStruct((N, D), jnp.float32),
          jax.ShapeDtypeStruct((1, D), jnp.float32),
      ],
  )(h, agg2[0], agg2[1], WnT, WnB, bn.reshape(1, D), g.reshape(1, D),
    be.reshape(1, D))


def kernel(x, edge_index, Wm0, bm0, Wn0, bn0, g0, be0, Wm1, bm1, Wn1, bn1,
           g1, be1):
  src_w = edge_index[0].reshape(NW, NSEC, SEC, CHUNK)
  dst_w = edge_index[1].reshape(NW, NSEC, SEC, CHUNK)

  def hop(h, Wm, bm, Wn, bn, g, be):
    P = _tc_msg(h, Wm, bm)
    agg2 = _sc_segment_sum(P, src_w, dst_w)[:, :N, :]
    return _tc_update(h, agg2, Wn[:D], Wn[D:], bn, g, be)

  h1, _ = hop(x, Wm0, bm0, Wn0, bn0, g0, be0)
  h2, ctx = hop(h1, Wm1, bm1, Wn1, bn1, g1, be1)
  return (h2, ctx)


# speedup vs baseline: 11.1795x; 1.1080x over previous
import functools

import jax
import jax.numpy as jnp
from jax import lax
from jax.experimental import pallas as pl
from jax.experimental.pallas import tpu as pltpu
from jax.experimental.pallas import tpu_sc as plsc

N = 10000
E = 320000
D = 128
EPS = 1e-3

NC = 2
NS = 16
NW = NC * NS
E_W = E // NW
CHUNK = 80
NCHUNK = E_W // CHUNK
NSEC = 5
SEC = NCHUNK // NSEC
NGRP = (SEC - 1) // 3
N_PAD = 10240
ROWS_TILE = N_PAD // NS

BM = 1000


def _sc_segment_sum(P, src_w, dst_w):
  mesh = plsc.VectorSubcoreMesh(core_axis_name="c", subcore_axis_name="s")

  @functools.partial(
      pl.kernel,
      mesh=mesh,
      out_type=jax.ShapeDtypeStruct((NC, N_PAD, D), jnp.float32),
      scratch_types=[
          pltpu.VMEM((SEC, CHUNK), jnp.int32),
          pltpu.VMEM((SEC, CHUNK), jnp.int32),
          pltpu.VMEM((CHUNK, D), jnp.float32),
          pltpu.VMEM((CHUNK, D), jnp.float32),
          pltpu.VMEM((CHUNK, D), jnp.float32),
          pltpu.VMEM_SHARED((N_PAD, D), jnp.float32),
          pltpu.SemaphoreType.DMA,
          pltpu.SemaphoreType.DMA,
          pltpu.SemaphoreType.DMA,
          pltpu.SemaphoreType.DMA,
          pltpu.SemaphoreType.DMA,
          pltpu.SemaphoreType.DMA,
      ],
  )
  def seg_sum(p_hbm, src_hbm, dst_hbm, out_hbm, src_v, dst_v, rows0, rows1,
              rows2, agg_sh, gsem0, gsem1, gsem2, ssem0, ssem1, ssem2):
    rows = (rows0, rows1, rows2)
    gsem = (gsem0, gsem1, gsem2)
    ssem = (ssem0, ssem1, ssem2)
    cid = lax.axis_index("c")
    sid = lax.axis_index("s")
    wid = sid * NC + cid

    def _zrow(i, _):
      r = i // (D // 16)
      c0 = (i % (D // 16)) * 16
      rows0[r, pl.ds(c0, 16)] = jnp.zeros((16,), jnp.float32)
      return 0
    lax.fori_loop(0, CHUNK * (D // 16), _zrow, 0)

    def _zcp(j, _):
      pltpu.sync_copy(rows0, agg_sh.at[pl.ds(sid * ROWS_TILE + j * CHUNK, CHUNK)])
      return 0
    lax.fori_loop(0, ROWS_TILE // CHUNK, _zcp, 0)
    plsc.subcore_barrier()

    def _wait_g(c, k):
      pltpu.make_async_copy(p_hbm.at[src_v.at[c]], rows[k], gsem[k]).wait()

    def _issue_s(c, k):
      pltpu.async_copy(rows[k], agg_sh.at[dst_v.at[c]], ssem[k], add=True)

    def _wait_s(c, k):
      pltpu.make_async_copy(rows[k], agg_sh.at[dst_v.at[c]], ssem[k]).wait()

    def _issue_g(c, k):
      pltpu.async_copy(p_hbm.at[src_v.at[c]], rows[k], gsem[k])

    for sec in range(NSEC):
      pltpu.sync_copy(src_hbm.at[wid, sec], src_v)
      pltpu.sync_copy(dst_hbm.at[wid, sec], dst_v)
      _issue_g(0, 0)
      _issue_g(1, 1)

      def _group(g, _):
        for k in range(3):
          c = 3 * g + k
          kp = (k + 2) % 3
          _wait_g(c, k)
          _issue_s(c, k)
          if k == 0:
            @pl.when(g > 0)
            def _w0():
              _wait_s(c, kp)
          else:
            _wait_s(c, kp)
          if k == 2:
            @pl.when(g + 1 < NGRP)
            def _g2():
              _issue_g(c + 2, kp)
          else:
            _issue_g(c + 2, kp)
        return 0

      lax.fori_loop(0, NGRP, _group, 0)
      _wait_g(SEC - 1, 0)
      _issue_s(SEC - 1, 0)
      _wait_s(SEC - 2, 2)
      _wait_s(SEC - 1, 0)
    plsc.subcore_barrier()

    pltpu.sync_copy(
        agg_sh.at[pl.ds(sid * ROWS_TILE, ROWS_TILE)],
        out_hbm.at[cid, pl.ds(sid * ROWS_TILE, ROWS_TILE)])

  return seg_sum(P, src_w, dst_w)


def _tc_msg(h, Wm, bm):
  def body(h_ref, w_ref, b_ref, o_ref):
    o_ref[...] = jnp.maximum(
        jnp.dot(h_ref[...], w_ref[...], preferred_element_type=jnp.float32)
        + b_ref[...], 0.0)

  return pl.pallas_call(
      body,
      grid=(N // BM,),
      in_specs=[
          pl.BlockSpec((BM, D), lambda i: (i, 0)),
          pl.BlockSpec((D, D), lambda i: (0, 0)),
          pl.BlockSpec((1, D), lambda i: (0, 0)),
      ],
      out_specs=pl.BlockSpec((BM, D), lambda i: (i, 0)),
      out_shape=jax.ShapeDtypeStruct((N, D), jnp.float32),
  )(h, Wm, bm.reshape(1, D))


def _tc_update(h, agg2, WnT, WnB, bn, g, be):
  def body(h_ref, a0_ref, a1_ref, wt_ref, wb_ref, b_ref, g_ref, be_ref,
           o_ref, ctx_ref):
    hv = h_ref[...]
    agg = a0_ref[...] + a1_ref[...]
    t = (jnp.dot(hv, wt_ref[...], preferred_element_type=jnp.float32)
         + jnp.dot(agg, wb_ref[...], preferred_element_type=jnp.float32)
         + b_ref[...])
    hn = jnp.maximum(hv + jnp.maximum(t, 0.0), 0.0)
    mu = jnp.mean(hn, axis=1, keepdims=True)
    var = jnp.mean((hn - mu) ** 2, axis=1, keepdims=True)
    y = (hn - mu) * lax.rsqrt(var + EPS) * g_ref[...] + be_ref[...]
    o_ref[...] = y

    @pl.when(pl.program_id(0) == 0)
    def _init():
      ctx_ref[...] = jnp.zeros_like(ctx_ref)

    ctx_ref[...] += jnp.sum(y, axis=0, keepdims=True)

    @pl.when(pl.program_id(0) == N // BM - 1)
    def _fin():
      ctx_ref[...] = ctx_ref[...] * (1.0 / N)

  return pl.pallas_call(
      body,
      grid=(N // BM,),
      in_specs=[
          pl.BlockSpec((BM, D), lambda i: (i, 0)),
          pl.BlockSpec((BM, D), lambda i: (i, 0)),
          pl.BlockSpec((BM, D), lambda i: (i, 0)),
          pl.BlockSpec((D, D), lambda i: (0, 0)),
          pl.BlockSpec((D, D), lambda i: (0, 0)),
          pl.BlockSpec((1, D), lambda i: (0, 0)),
          pl.BlockSpec((1, D), lambda i: (0, 0)),
          pl.BlockSpec((1, D), lambda i: (0, 0)),
      ],
      out_specs=[
          pl.BlockSpec((BM, D), lambda i: (i, 0)),
          pl.BlockSpec((1, D), lambda i: (0, 0)),
      ],
      out_shape=[
          jax.ShapeDtypeStruct((N, D), jnp.float32),
          jax.ShapeDtypeStruct((1, D), jnp.float32),
      ],
  )(h, agg2[0], agg2[1], WnT, WnB, bn.reshape(1, D), g.reshape(1, D),
    be.reshape(1, D))


def kernel(x, edge_index, Wm0, bm0, Wn0, bn0, g0, be0, Wm1, bm1, Wn1, bn1,
           g1, be1):
  src_w = edge_index[0].reshape(NW, NSEC, SEC, CHUNK)
  dst_w = edge_index[1].reshape(NW, NSEC, SEC, CHUNK)

  def hop(h, Wm, bm, Wn, bn, g, be):
    P = _tc_msg(h, Wm, bm)
    agg2 = _sc_segment_sum(P, src_w, dst_w)[:, :N, :]
    return _tc_update(h, agg2, Wn[:D], Wn[D:], bn, g, be)

  h1, _ = hop(x, Wm0, bm0, Wn0, bn0, g0, be0)
  h2, ctx = hop(h1, Wm1, bm1, Wn1, bn1, g1, be1)
  return (h2, ctx)


# speedup vs baseline: 12.0532x; 1.0782x over previous
import functools

import jax
import jax.numpy as jnp
from jax import lax
from jax.experimental import pallas as pl
from jax.experimental.pallas import tpu as pltpu
from jax.experimental.pallas import tpu_sc as plsc

N = 10000
E = 320000
D = 128
EPS = 1e-3

NC = 2
NS = 16
NW = NC * NS
E_W = E // NW
CHUNK = 80
NCHUNK = E_W // CHUNK
NSEC = 5
SEC = NCHUNK // NSEC
NGRP = (SEC - 1) // 3
N_PAD = 10240
ROWS_TILE = N_PAD // NS

BM = 1000


def _sc_segment_sum(P, src_w, dst_w):
  mesh = plsc.VectorSubcoreMesh(core_axis_name="c", subcore_axis_name="s")

  @functools.partial(
      pl.kernel,
      mesh=mesh,
      out_type=jax.ShapeDtypeStruct((NC, N_PAD, D), jnp.float32),
      scratch_types=[
          pltpu.VMEM((SEC, CHUNK), jnp.int32),
          pltpu.VMEM((SEC, CHUNK), jnp.int32),
          pltpu.VMEM((CHUNK, D), jnp.float32),
          pltpu.VMEM((CHUNK, D), jnp.float32),
          pltpu.VMEM((CHUNK, D), jnp.float32),
          pltpu.VMEM_SHARED((N_PAD, D), jnp.float32),
          pltpu.SemaphoreType.DMA,
          pltpu.SemaphoreType.DMA,
          pltpu.SemaphoreType.DMA,
          pltpu.SemaphoreType.DMA,
          pltpu.SemaphoreType.DMA,
          pltpu.SemaphoreType.DMA,
      ],
  )
  def seg_sum(p_hbm, src_hbm, dst_hbm, out_hbm, src_v, dst_v, rows0, rows1,
              rows2, agg_sh, gsem0, gsem1, gsem2, ssem0, ssem1, ssem2):
    rows = (rows0, rows1, rows2)
    gsem = (gsem0, gsem1, gsem2)
    ssem = (ssem0, ssem1, ssem2)
    cid = lax.axis_index("c")
    sid = lax.axis_index("s")
    wid = sid * NC + cid

    def _zrow(i, _):
      r = i // (D // 16)
      c0 = (i % (D // 16)) * 16
      rows0[r, pl.ds(c0, 16)] = jnp.zeros((16,), jnp.float32)
      return 0
    lax.fori_loop(0, CHUNK * (D // 16), _zrow, 0)

    def _zcp(j, _):
      pltpu.sync_copy(rows0, agg_sh.at[pl.ds(sid * ROWS_TILE + j * CHUNK, CHUNK)])
      return 0
    lax.fori_loop(0, ROWS_TILE // CHUNK, _zcp, 0)
    plsc.subcore_barrier()

    def _wait_g(c, k):
      pltpu.make_async_copy(p_hbm.at[src_v.at[c]], rows[k], gsem[k]).wait()

    def _issue_s(c, k):
      pltpu.async_copy(rows[k], agg_sh.at[dst_v.at[c]], ssem[k], add=True)

    def _wait_s(c, k):
      pltpu.make_async_copy(rows[k], agg_sh.at[dst_v.at[c]], ssem[k]).wait()

    def _issue_g(c, k):
      pltpu.async_copy(p_hbm.at[src_v.at[c]], rows[k], gsem[k])

    for sec in range(NSEC):
      pltpu.sync_copy(src_hbm.at[wid, sec], src_v)
      pltpu.sync_copy(dst_hbm.at[wid, sec], dst_v)
      _issue_g(0, 0)
      _issue_g(1, 1)

      def _group(g, _):
        for k in range(3):
          c = 3 * g + k
          kp = (k + 2) % 3
          _wait_g(c, k)
          _issue_s(c, k)
          if k == 0:
            @pl.when(g > 0)
            def _w0():
              _wait_s(c, kp)
          else:
            _wait_s(c, kp)
          if k == 2:
            @pl.when(g + 1 < NGRP)
            def _g2():
              _issue_g(c + 2, kp)
          else:
            _issue_g(c + 2, kp)
        return 0

      lax.fori_loop(0, NGRP, _group, 0)
      _wait_g(SEC - 1, 0)
      _issue_s(SEC - 1, 0)
      _wait_s(SEC - 2, 2)
      _wait_s(SEC - 1, 0)
    plsc.subcore_barrier()

    pltpu.sync_copy(
        agg_sh.at[pl.ds(sid * ROWS_TILE, ROWS_TILE)],
        out_hbm.at[cid, pl.ds(sid * ROWS_TILE, ROWS_TILE)])

  return seg_sum(P, src_w, dst_w)


def _tc_msg(h, Wm, bm):
  def body(h_ref, w_ref, b_ref, o_ref):
    o_ref[...] = jnp.maximum(
        jnp.dot(h_ref[...], w_ref[...], preferred_element_type=jnp.float32)
        + b_ref[...], 0.0)

  return pl.pallas_call(
      body,
      grid=(N // BM,),
      in_specs=[
          pl.BlockSpec((BM, D), lambda i: (i, 0)),
          pl.BlockSpec((D, D), lambda i: (0, 0)),
          pl.BlockSpec((1, D), lambda i: (0, 0)),
      ],
      out_specs=pl.BlockSpec((BM, D), lambda i: (i, 0)),
      out_shape=jax.ShapeDtypeStruct((N, D), jnp.float32),
  )(h, Wm, bm.reshape(1, D))


def _update_block(hv, a0, a1, wt, wb, b, gg, be):
  t = (jnp.dot(hv, wt, preferred_element_type=jnp.float32)
       + jnp.dot(a0 + a1, wb, preferred_element_type=jnp.float32) + b)
  hn = jnp.maximum(hv + jnp.maximum(t, 0.0), 0.0)
  mu = jnp.mean(hn, axis=1, keepdims=True)
  var = jnp.mean((hn - mu) ** 2, axis=1, keepdims=True)
  return (hn - mu) * lax.rsqrt(var + EPS) * gg + be


_AGG_SPECS = [
    pl.BlockSpec((BM, D), lambda i: (i, 0)),
    pl.BlockSpec((1, BM, D), lambda i: (0, i, 0)),
    pl.BlockSpec((1, BM, D), lambda i: (1, i, 0)),
    pl.BlockSpec((D, D), lambda i: (0, 0)),
    pl.BlockSpec((D, D), lambda i: (0, 0)),
    pl.BlockSpec((1, D), lambda i: (0, 0)),
    pl.BlockSpec((1, D), lambda i: (0, 0)),
    pl.BlockSpec((1, D), lambda i: (0, 0)),
]


def _tc_update_msg(h, agg2, WnT, WnB, bn, g, be, Wm_next, bm_next):
  def body(h_ref, a0_ref, a1_ref, wt_ref, wb_ref, b_ref, g_ref, be_ref,
           wm_ref, bm_ref, o_ref, p_ref):
    y = _update_block(h_ref[...], a0_ref[0], a1_ref[0], wt_ref[...],
                      wb_ref[...], b_ref[...], g_ref[...], be_ref[...])
    o_ref[...] = y
    p_ref[...] = jnp.maximum(
        jnp.dot(y, wm_ref[...], preferred_element_type=jnp.float32)
        + bm_ref[...], 0.0)

  return pl.pallas_call(
      body,
      grid=(N // BM,),
      in_specs=_AGG_SPECS + [
          pl.BlockSpec((D, D), lambda i: (0, 0)),
          pl.BlockSpec((1, D), lambda i: (0, 0)),
      ],
      out_specs=[
          pl.BlockSpec((BM, D), lambda i: (i, 0)),
          pl.BlockSpec((BM, D), lambda i: (i, 0)),
      ],
      out_shape=[
          jax.ShapeDtypeStruct((N, D), jnp.float32),
          jax.ShapeDtypeStruct((N, D), jnp.float32),
      ],
  )(h, agg2, agg2, WnT, WnB, bn.reshape(1, D), g.reshape(1, D),
    be.reshape(1, D), Wm_next, bm_next.reshape(1, D))


def _tc_update_final(h, agg2, WnT, WnB, bn, g, be):
  def body(h_ref, a0_ref, a1_ref, wt_ref, wb_ref, b_ref, g_ref, be_ref,
           o_ref, ctx_ref):
    y = _update_block(h_ref[...], a0_ref[0], a1_ref[0], wt_ref[...],
                      wb_ref[...], b_ref[...], g_ref[...], be_ref[...])
    o_ref[...] = y

    @pl.when(pl.program_id(0) == 0)
    def _init():
      ctx_ref[...] = jnp.zeros_like(ctx_ref)

    ctx_ref[...] += jnp.sum(y, axis=0, keepdims=True)

    @pl.when(pl.program_id(0) == N // BM - 1)
    def _fin():
      ctx_ref[...] = ctx_ref[...] * (1.0 / N)

  return pl.pallas_call(
      body,
      grid=(N // BM,),
      in_specs=_AGG_SPECS,
      out_specs=[
          pl.BlockSpec((BM, D), lambda i: (i, 0)),
          pl.BlockSpec((1, D), lambda i: (0, 0)),
      ],
      out_shape=[
          jax.ShapeDtypeStruct((N, D), jnp.float32),
          jax.ShapeDtypeStruct((1, D), jnp.float32),
      ],
  )(h, agg2, agg2, WnT, WnB, bn.reshape(1, D), g.reshape(1, D),
    be.reshape(1, D))


def kernel(x, edge_index, Wm0, bm0, Wn0, bn0, g0, be0, Wm1, bm1, Wn1, bn1,
           g1, be1):
  src_w = edge_index[0].reshape(NW, NSEC, SEC, CHUNK)
  dst_w = edge_index[1].reshape(NW, NSEC, SEC, CHUNK)

  P0 = _tc_msg(x, Wm0, bm0)
  agg0 = _sc_segment_sum(P0, src_w, dst_w)
  h1, P1 = _tc_update_msg(x, agg0, Wn0[:D], Wn0[D:], bn0, g0, be0, Wm1, bm1)
  agg1 = _sc_segment_sum(P1, src_w, dst_w)
  h2, ctx = _tc_update_final(h1, agg1, Wn1[:D], Wn1[D:], bn1, g1, be1)
  return (h2, ctx)


# speedup vs baseline: 12.3357x; 1.0234x over previous
import functools

import jax
import jax.numpy as jnp
from jax import lax
from jax.experimental import pallas as pl
from jax.experimental.pallas import tpu as pltpu
from jax.experimental.pallas import tpu_sc as plsc

N = 10000
E = 320000
D = 128
EPS = 1e-3

NC = 2
NS = 16
NW = NC * NS
E_W = E // NW
CHUNK = 80
NCHUNK = E_W // CHUNK
NSEC = 5
SEC = NCHUNK // NSEC
NGRP = (SEC - 1) // 3
N_PAD = 10240
ROWS_TILE = N_PAD // NS

BM = 1000


def _sc_segment_sum(P, src_w, dst_w):
  mesh = plsc.VectorSubcoreMesh(core_axis_name="c", subcore_axis_name="s")

  @functools.partial(
      pl.kernel,
      mesh=mesh,
      out_type=jax.ShapeDtypeStruct((NC, N_PAD, D), jnp.float32),
      scratch_types=[
          pltpu.VMEM((SEC, CHUNK), jnp.int32),
          pltpu.VMEM((SEC, CHUNK), jnp.int32),
          pltpu.VMEM((CHUNK, D), jnp.float32),
          pltpu.VMEM((CHUNK, D), jnp.float32),
          pltpu.VMEM((CHUNK, D), jnp.float32),
          pltpu.VMEM_SHARED((N_PAD, D), jnp.float32),
          pltpu.SemaphoreType.DMA,
          pltpu.SemaphoreType.DMA,
          pltpu.SemaphoreType.DMA,
          pltpu.SemaphoreType.DMA,
          pltpu.SemaphoreType.DMA,
          pltpu.SemaphoreType.DMA,
      ],
  )
  def seg_sum(p_hbm, src_hbm, dst_hbm, out_hbm, src_v, dst_v, rows0, rows1,
              rows2, agg_sh, gsem0, gsem1, gsem2, ssem0, ssem1, ssem2):
    rows = (rows0, rows1, rows2)
    gsem = (gsem0, gsem1, gsem2)
    ssem = (ssem0, ssem1, ssem2)
    cid = lax.axis_index("c")
    sid = lax.axis_index("s")
    wid = sid * NC + cid

    def _wait_g(c, k):
      pltpu.make_async_copy(p_hbm.at[src_v.at[c]], rows[k], gsem[k]).wait()

    def _issue_s(c, k):
      pltpu.async_copy(rows[k], agg_sh.at[dst_v.at[c]], ssem[k], add=True)

    def _wait_s(c, k):
      pltpu.make_async_copy(rows[k], agg_sh.at[dst_v.at[c]], ssem[k]).wait()

    def _issue_g(c, k):
      pltpu.async_copy(p_hbm.at[src_v.at[c]], rows[k], gsem[k])

    pltpu.sync_copy(src_hbm.at[wid, 0], src_v)
    pltpu.sync_copy(dst_hbm.at[wid, 0], dst_v)
    _issue_g(0, 0)
    _issue_g(1, 1)

    def _zrow(i, _):
      r = i // (D // 16)
      c0 = (i % (D // 16)) * 16
      rows2[r, pl.ds(c0, 16)] = jnp.zeros((16,), jnp.float32)
      return 0
    lax.fori_loop(0, CHUNK * (D // 16), _zrow, 0)

    def _zcp(j, _):
      pltpu.sync_copy(rows2, agg_sh.at[pl.ds(sid * ROWS_TILE + j * CHUNK, CHUNK)])
      return 0
    lax.fori_loop(0, ROWS_TILE // CHUNK, _zcp, 0)
    plsc.subcore_barrier()

    for sec in range(NSEC):
      def _group(g, _):
        for k in range(3):
          c = 3 * g + k
          kp = (k + 2) % 3
          _wait_g(c, k)
          _issue_s(c, k)
          if k == 0:
            @pl.when(g > 0)
            def _w0():
              _wait_s(c, kp)
          else:
            _wait_s(c, kp)
          if k == 2:
            @pl.when(g + 1 < NGRP)
            def _g2():
              _issue_g(c + 2, kp)
          else:
            _issue_g(c + 2, kp)
        return 0

      lax.fori_loop(0, NGRP, _group, 0)
      _wait_g(SEC - 1, 0)
      _issue_s(SEC - 1, 0)
      if sec + 1 < NSEC:
        pltpu.sync_copy(src_hbm.at[wid, sec + 1], src_v)
        _issue_g(1, 1)
      _wait_s(SEC - 2, 2)
      _wait_s(SEC - 1, 0)
      if sec + 1 < NSEC:
        pltpu.sync_copy(dst_hbm.at[wid, sec + 1], dst_v)
        _issue_g(0, 0)
    plsc.subcore_barrier()

    pltpu.sync_copy(
        agg_sh.at[pl.ds(sid * ROWS_TILE, ROWS_TILE)],
        out_hbm.at[cid, pl.ds(sid * ROWS_TILE, ROWS_TILE)])

  return seg_sum(P, src_w, dst_w)


def _tc_msg(h, Wm, bm):
  def body(h_ref, w_ref, b_ref, o_ref):
    o_ref[...] = jnp.maximum(
        jnp.dot(h_ref[...], w_ref[...], preferred_element_type=jnp.float32)
        + b_ref[...], 0.0)

  return pl.pallas_call(
      body,
      grid=(N // BM,),
      in_specs=[
          pl.BlockSpec((BM, D), lambda i: (i, 0)),
          pl.BlockSpec((D, D), lambda i: (0, 0)),
          pl.BlockSpec((1, D), lambda i: (0, 0)),
      ],
      out_specs=pl.BlockSpec((BM, D), lambda i: (i, 0)),
      out_shape=jax.ShapeDtypeStruct((N, D), jnp.float32),
  )(h, Wm, bm.reshape(1, D))


def _update_block(hv, a0, a1, wt, wb, b, gg, be):
  t = (jnp.dot(hv, wt, preferred_element_type=jnp.float32)
       + jnp.dot(a0 + a1, wb, preferred_element_type=jnp.float32) + b)
  hn = jnp.maximum(hv + jnp.maximum(t, 0.0), 0.0)
  mu = jnp.mean(hn, axis=1, keepdims=True)
  var = jnp.mean((hn - mu) ** 2, axis=1, keepdims=True)
  return (hn - mu) * lax.rsqrt(var + EPS) * gg + be


_AGG_SPECS = [
    pl.BlockSpec((BM, D), lambda i: (i, 0)),
    pl.BlockSpec((1, BM, D), lambda i: (0, i, 0)),
    pl.BlockSpec((1, BM, D), lambda i: (1, i, 0)),
    pl.BlockSpec((D, D), lambda i: (0, 0)),
    pl.BlockSpec((D, D), lambda i: (0, 0)),
    pl.BlockSpec((1, D), lambda i: (0, 0)),
    pl.BlockSpec((1, D), lambda i: (0, 0)),
    pl.BlockSpec((1, D), lambda i: (0, 0)),
]


def _tc_update_msg(h, agg2, WnT, WnB, bn, g, be, Wm_next, bm_next):
  def body(h_ref, a0_ref, a1_ref, wt_ref, wb_ref, b_ref, g_ref, be_ref,
           wm_ref, bm_ref, o_ref, p_ref):
    y = _update_block(h_ref[...], a0_ref[0], a1_ref[0], wt_ref[...],
                      wb_ref[...], b_ref[...], g_ref[...], be_ref[...])
    o_ref[...] = y
    p_ref[...] = jnp.maximum(
        jnp.dot(y, wm_ref[...], preferred_element_type=jnp.float32)
        + bm_ref[...], 0.0)

  return pl.pallas_call(
      body,
      grid=(N // BM,),
      in_specs=_AGG_SPECS + [
          pl.BlockSpec((D, D), lambda i: (0, 0)),
          pl.BlockSpec((1, D), lambda i: (0, 0)),
      ],
      out_specs=[
          pl.BlockSpec((BM, D), lambda i: (i, 0)),
          pl.BlockSpec((BM, D), lambda i: (i, 0)),
      ],
      out_shape=[
          jax.ShapeDtypeStruct((N, D), jnp.float32),
          jax.ShapeDtypeStruct((N, D), jnp.float32),
      ],
  )(h, agg2, agg2, WnT, WnB, bn.reshape(1, D), g.reshape(1, D),
    be.reshape(1, D), Wm_next, bm_next.reshape(1, D))


def _tc_update_final(h, agg2, WnT, WnB, bn, g, be):
  def body(h_ref, a0_ref, a1_ref, wt_ref, wb_ref, b_ref, g_ref, be_ref,
           o_ref, ctx_ref):
    y = _update_block(h_ref[...], a0_ref[0], a1_ref[0], wt_ref[...],
                      wb_ref[...], b_ref[...], g_ref[...], be_ref[...])
    o_ref[...] = y

    @pl.when(pl.program_id(0) == 0)
    def _init():
      ctx_ref[...] = jnp.zeros_like(ctx_ref)

    ctx_ref[...] += jnp.sum(y, axis=0, keepdims=True)

    @pl.when(pl.program_id(0) == N // BM - 1)
    def _fin():
      ctx_ref[...] = ctx_ref[...] * (1.0 / N)

  return pl.pallas_call(
      body,
      grid=(N // BM,),
      in_specs=_AGG_SPECS,
      out_specs=[
          pl.BlockSpec((BM, D), lambda i: (i, 0)),
          pl.BlockSpec((1, D), lambda i: (0, 0)),
      ],
      out_shape=[
          jax.ShapeDtypeStruct((N, D), jnp.float32),
          jax.ShapeDtypeStruct((1, D), jnp.float32),
      ],
  )(h, agg2, agg2, WnT, WnB, bn.reshape(1, D), g.reshape(1, D),
    be.reshape(1, D))


def kernel(x, edge_index, Wm0, bm0, Wn0, bn0, g0, be0, Wm1, bm1, Wn1, bn1,
           g1, be1):
  src_w = edge_index[0].reshape(NW, NSEC, SEC, CHUNK)
  dst_w = edge_index[1].reshape(NW, NSEC, SEC, CHUNK)

  P0 = _tc_msg(x, Wm0, bm0)
  agg0 = _sc_segment_sum(P0, src_w, dst_w)
  h1, P1 = _tc_update_msg(x, agg0, Wn0[:D], Wn0[D:], bn0, g0, be0, Wm1, bm1)
  agg1 = _sc_segment_sum(P1, src_w, dst_w)
  h2, ctx = _tc_update_final(h1, agg1, Wn1[:D], Wn1[D:], bn1, g1, be1)
  return (h2, ctx)


# speedup vs baseline: 13.2405x; 1.0733x over previous
import functools

import jax
import jax.numpy as jnp
from jax import lax
from jax.experimental import pallas as pl
from jax.experimental.pallas import tpu as pltpu
from jax.experimental.pallas import tpu_sc as plsc

N = 10000
E = 320000
D = 128
EPS = 1e-3

NC = 2
NS = 16
NW = NC * NS
E_W = E // NW
CHUNK = 80
NCHUNK = E_W // CHUNK
NSEC = 5
SEC = NCHUNK // NSEC
NGRP = (SEC - 1) // 3
N_PAD = 10240
ROWS_TILE = N_PAD // NS

BM = 2000


def _sc_segment_sum(P, edges_w):
  mesh = plsc.VectorSubcoreMesh(core_axis_name="c", subcore_axis_name="s")

  @functools.partial(
      pl.kernel,
      mesh=mesh,
      out_type=jax.ShapeDtypeStruct((NC, N_PAD, D), jnp.float32),
      scratch_types=[
          pltpu.VMEM((SEC, CHUNK), jnp.int32),
          pltpu.VMEM((SEC, CHUNK), jnp.int32),
          pltpu.VMEM((CHUNK, D), jnp.float32),
          pltpu.VMEM((CHUNK, D), jnp.float32),
          pltpu.VMEM((CHUNK, D), jnp.float32),
          pltpu.VMEM_SHARED((N_PAD, D), jnp.float32),
          pltpu.SemaphoreType.DMA,
          pltpu.SemaphoreType.DMA,
          pltpu.SemaphoreType.DMA,
          pltpu.SemaphoreType.DMA,
          pltpu.SemaphoreType.DMA,
          pltpu.SemaphoreType.DMA,
      ],
  )
  def seg_sum(p_hbm, e_hbm, out_hbm, src_v, dst_v, rows0, rows1,
              rows2, agg_sh, gsem0, gsem1, gsem2, ssem0, ssem1, ssem2):
    rows = (rows0, rows1, rows2)
    gsem = (gsem0, gsem1, gsem2)
    ssem = (ssem0, ssem1, ssem2)
    cid = lax.axis_index("c")
    sid = lax.axis_index("s")
    wid = sid * NC + cid

    def _wait_g(c, k):
      pltpu.make_async_copy(p_hbm.at[src_v.at[c]], rows[k], gsem[k]).wait()

    def _issue_s(c, k):
      pltpu.async_copy(rows[k], agg_sh.at[dst_v.at[c]], ssem[k], add=True)

    def _wait_s(c, k):
      pltpu.make_async_copy(rows[k], agg_sh.at[dst_v.at[c]], ssem[k]).wait()

    def _issue_g(c, k):
      pltpu.async_copy(p_hbm.at[src_v.at[c]], rows[k], gsem[k])

    pltpu.sync_copy(e_hbm.at[0, wid, 0], src_v)
    pltpu.sync_copy(e_hbm.at[1, wid, 0], dst_v)
    _issue_g(0, 0)
    _issue_g(1, 1)

    def _zrow(i, _):
      r = i // (D // 16)
      c0 = (i % (D // 16)) * 16
      rows2[r, pl.ds(c0, 16)] = jnp.zeros((16,), jnp.float32)
      return 0
    lax.fori_loop(0, CHUNK * (D // 16), _zrow, 0)

    def _zcp(j, _):
      pltpu.sync_copy(rows2, agg_sh.at[pl.ds(sid * ROWS_TILE + j * CHUNK, CHUNK)])
      return 0
    lax.fori_loop(0, ROWS_TILE // CHUNK, _zcp, 0)
    plsc.subcore_barrier()

    for sec in range(NSEC):
      def _group(g, _):
        for k in range(3):
          c = 3 * g + k
          kp = (k + 2) % 3
          _wait_g(c, k)
          _issue_s(c, k)
          if k == 0:
            @pl.when(g > 0)
            def _w0():
              _wait_s(c, kp)
          else:
            _wait_s(c, kp)
          if k == 2:
            @pl.when(g + 1 < NGRP)
            def _g2():
              _issue_g(c + 2, kp)
          else:
            _issue_g(c + 2, kp)
        return 0

      lax.fori_loop(0, NGRP, _group, 0)
      _wait_g(SEC - 1, 0)
      _issue_s(SEC - 1, 0)
      if sec + 1 < NSEC:
        pltpu.sync_copy(e_hbm.at[0, wid, sec + 1], src_v)
        _issue_g(1, 1)
      _wait_s(SEC - 2, 2)
      _wait_s(SEC - 1, 0)
      if sec + 1 < NSEC:
        pltpu.sync_copy(e_hbm.at[1, wid, sec + 1], dst_v)
        _issue_g(0, 0)
    plsc.subcore_barrier()

    pltpu.sync_copy(
        agg_sh.at[pl.ds(sid * ROWS_TILE, ROWS_TILE)],
        out_hbm.at[cid, pl.ds(sid * ROWS_TILE, ROWS_TILE)])

  return seg_sum(P, edges_w)


def _tc_msg(h, Wm, bm):
  def body(h_ref, w_ref, b_ref, o_ref):
    o_ref[...] = jnp.maximum(
        jnp.dot(h_ref[...], w_ref[...], preferred_element_type=jnp.float32)
        + b_ref[...], 0.0)

  return pl.pallas_call(
      body,
      grid=(N // BM,),
      in_specs=[
          pl.BlockSpec((BM, D), lambda i: (i, 0)),
          pl.BlockSpec((D, D), lambda i: (0, 0)),
          pl.BlockSpec((1, D), lambda i: (0, 0)),
      ],
      out_specs=pl.BlockSpec((BM, D), lambda i: (i, 0)),
      out_shape=jax.ShapeDtypeStruct((N, D), jnp.float32),
  )(h, Wm, bm.reshape(1, D))


def _update_block(hv, a0, a1, wt, wb, b, gg, be):
  t = (jnp.dot(hv, wt, preferred_element_type=jnp.float32)
       + jnp.dot(a0 + a1, wb, preferred_element_type=jnp.float32) + b)
  hn = jnp.maximum(hv + jnp.maximum(t, 0.0), 0.0)
  mu = jnp.mean(hn, axis=1, keepdims=True)
  var = jnp.mean((hn - mu) ** 2, axis=1, keepdims=True)
  return (hn - mu) * lax.rsqrt(var + EPS) * gg + be


_AGG_SPECS = [
    pl.BlockSpec((BM, D), lambda i: (i, 0)),
    pl.BlockSpec((1, BM, D), lambda i: (0, i, 0)),
    pl.BlockSpec((1, BM, D), lambda i: (1, i, 0)),
    pl.BlockSpec((D, D), lambda i: (0, 0)),
    pl.BlockSpec((D, D), lambda i: (1, 0)),
    pl.BlockSpec((1, D), lambda i: (0, 0)),
    pl.BlockSpec((1, D), lambda i: (0, 0)),
    pl.BlockSpec((1, D), lambda i: (0, 0)),
]


def _tc_update_msg(h, agg2, Wn, bn, g, be, Wm_next, bm_next):
  def body(h_ref, a0_ref, a1_ref, wt_ref, wb_ref, b_ref, g_ref, be_ref,
           wm_ref, bm_ref, o_ref, p_ref):
    y = _update_block(h_ref[...], a0_ref[0], a1_ref[0], wt_ref[...],
                      wb_ref[...], b_ref[...], g_ref[...], be_ref[...])
    o_ref[...] = y
    p_ref[...] = jnp.maximum(
        jnp.dot(y, wm_ref[...], preferred_element_type=jnp.float32)
        + bm_ref[...], 0.0)

  return pl.pallas_call(
      body,
      grid=(N // BM,),
      in_specs=_AGG_SPECS + [
          pl.BlockSpec((D, D), lambda i: (0, 0)),
          pl.BlockSpec((1, D), lambda i: (0, 0)),
      ],
      out_specs=[
          pl.BlockSpec((BM, D), lambda i: (i, 0)),
          pl.BlockSpec((BM, D), lambda i: (i, 0)),
      ],
      out_shape=[
          jax.ShapeDtypeStruct((N, D), jnp.float32),
          jax.ShapeDtypeStruct((N, D), jnp.float32),
      ],
  )(h, agg2, agg2, Wn, Wn, bn.reshape(1, D), g.reshape(1, D),
    be.reshape(1, D), Wm_next, bm_next.reshape(1, D))


def _tc_update_final(h, agg2, Wn, bn, g, be):
  def body(h_ref, a0_ref, a1_ref, wt_ref, wb_ref, b_ref, g_ref, be_ref,
           o_ref, ctx_ref):
    y = _update_block(h_ref[...], a0_ref[0], a1_ref[0], wt_ref[...],
                      wb_ref[...], b_ref[...], g_ref[...], be_ref[...])
    o_ref[...] = y

    @pl.when(pl.program_id(0) == 0)
    def _init():
      ctx_ref[...] = jnp.zeros_like(ctx_ref)

    ctx_ref[...] += jnp.sum(y, axis=0, keepdims=True)

    @pl.when(pl.program_id(0) == N // BM - 1)
    def _fin():
      ctx_ref[...] = ctx_ref[...] * (1.0 / N)

  return pl.pallas_call(
      body,
      grid=(N // BM,),
      in_specs=_AGG_SPECS,
      out_specs=[
          pl.BlockSpec((BM, D), lambda i: (i, 0)),
          pl.BlockSpec((1, D), lambda i: (0, 0)),
      ],
      out_shape=[
          jax.ShapeDtypeStruct((N, D), jnp.float32),
          jax.ShapeDtypeStruct((1, D), jnp.float32),
      ],
  )(h, agg2, agg2, Wn, Wn, bn.reshape(1, D), g.reshape(1, D),
    be.reshape(1, D))


def kernel(x, edge_index, Wm0, bm0, Wn0, bn0, g0, be0, Wm1, bm1, Wn1, bn1,
           g1, be1):
  edges_w = edge_index.reshape(2, NW, NSEC, SEC, CHUNK)

  P0 = _tc_msg(x, Wm0, bm0)
  agg0 = _sc_segment_sum(P0, edges_w)
  h1, P1 = _tc_update_msg(x, agg0, Wn0, bn0, g0, be0, Wm1, bm1)
  agg1 = _sc_segment_sum(P1, edges_w)
  h2, ctx = _tc_update_final(h1, agg1, Wn1, bn1, g1, be1)
  return (h2, ctx)


# speedup vs baseline: 13.3898x; 1.0113x over previous
import functools

import jax
import jax.numpy as jnp
from jax import lax
from jax.experimental import pallas as pl
from jax.experimental.pallas import tpu as pltpu
from jax.experimental.pallas import tpu_sc as plsc

N = 10000
E = 320000
D = 128
EPS = 1e-3

NC = 2
NS = 16
NW = NC * NS
E_W = E // NW
CHUNK = 80
NCHUNK = E_W // CHUNK
NSEC = 5
SEC = NCHUNK // NSEC
NGRP = (SEC - 1) // 3
N_PAD = 10240
ROWS_TILE = N_PAD // NS

BM = 2000


def _sc_segment_sum(P, edges_w):
  mesh = plsc.VectorSubcoreMesh(core_axis_name="c", subcore_axis_name="s")

  @functools.partial(
      pl.kernel,
      mesh=mesh,
      out_type=jax.ShapeDtypeStruct((NC, N_PAD, D), jnp.float32),
      scratch_types=[
          pltpu.VMEM((SEC * CHUNK,), jnp.int32),
          pltpu.VMEM((SEC * CHUNK,), jnp.int32),
          pltpu.VMEM((CHUNK, D), jnp.float32),
          pltpu.VMEM((CHUNK, D), jnp.float32),
          pltpu.VMEM((CHUNK, D), jnp.float32),
          pltpu.VMEM_SHARED((N_PAD, D), jnp.float32),
          pltpu.SemaphoreType.DMA,
          pltpu.SemaphoreType.DMA,
          pltpu.SemaphoreType.DMA,
          pltpu.SemaphoreType.DMA,
          pltpu.SemaphoreType.DMA,
          pltpu.SemaphoreType.DMA,
      ],
  )
  def seg_sum(p_hbm, e_hbm, out_hbm, src_v, dst_v, rows0, rows1,
              rows2, agg_sh, gsem0, gsem1, gsem2, ssem0, ssem1, ssem2):
    rows = (rows0, rows1, rows2)
    gsem = (gsem0, gsem1, gsem2)
    ssem = (ssem0, ssem1, ssem2)
    cid = lax.axis_index("c")
    sid = lax.axis_index("s")
    wid = sid * NC + cid

    def _wait_g(c, k):
      pltpu.make_async_copy(p_hbm.at[src_v.at[pl.ds(c * CHUNK, CHUNK)]],
                            rows[k], gsem[k]).wait()

    def _issue_s(c, k):
      pltpu.async_copy(rows[k], agg_sh.at[dst_v.at[pl.ds(c * CHUNK, CHUNK)]],
                       ssem[k], add=True)

    def _wait_s(c, k):
      pltpu.make_async_copy(rows[k],
                            agg_sh.at[dst_v.at[pl.ds(c * CHUNK, CHUNK)]],
                            ssem[k]).wait()

    def _issue_g(c, k):
      pltpu.async_copy(p_hbm.at[src_v.at[pl.ds(c * CHUNK, CHUNK)]],
                       rows[k], gsem[k])

    SECW = SEC * CHUNK

    pltpu.sync_copy(e_hbm.at[pl.ds(wid * E_W, SECW)], src_v)
    pltpu.sync_copy(e_hbm.at[pl.ds(E + wid * E_W, SECW)], dst_v)
    _issue_g(0, 0)
    _issue_g(1, 1)

    def _zrow(i, _):
      r = i // (D // 16)
      c0 = (i % (D // 16)) * 16
      rows2[r, pl.ds(c0, 16)] = jnp.zeros((16,), jnp.float32)
      return 0
    lax.fori_loop(0, CHUNK * (D // 16), _zrow, 0)

    def _zcp(j, _):
      pltpu.sync_copy(rows2, agg_sh.at[pl.ds(sid * ROWS_TILE + j * CHUNK, CHUNK)])
      return 0
    lax.fori_loop(0, ROWS_TILE // CHUNK, _zcp, 0)
    plsc.subcore_barrier()

    for sec in range(NSEC):
      def _group(g, _):
        for k in range(3):
          c = 3 * g + k
          kp = (k + 2) % 3
          _wait_g(c, k)
          _issue_s(c, k)
          if k == 0:
            @pl.when(g > 0)
            def _w0():
              _wait_s(c, kp)
          else:
            _wait_s(c, kp)
          if k == 2:
            @pl.when(g + 1 < NGRP)
            def _g2():
              _issue_g(c + 2, kp)
          else:
            _issue_g(c + 2, kp)
        return 0

      lax.fori_loop(0, NGRP, _group, 0)
      _wait_g(SEC - 1, 0)
      _issue_s(SEC - 1, 0)
      if sec + 1 < NSEC:
        pltpu.sync_copy(e_hbm.at[pl.ds(wid * E_W + (sec + 1) * SECW, SECW)],
                        src_v)
        _issue_g(1, 1)
      _wait_s(SEC - 2, 2)
      _wait_s(SEC - 1, 0)
      if sec + 1 < NSEC:
        pltpu.sync_copy(
            e_hbm.at[pl.ds(E + wid * E_W + (sec + 1) * SECW, SECW)], dst_v)
        _issue_g(0, 0)
    plsc.subcore_barrier()

    pltpu.sync_copy(
        agg_sh.at[pl.ds(sid * ROWS_TILE, ROWS_TILE)],
        out_hbm.at[cid, pl.ds(sid * ROWS_TILE, ROWS_TILE)])

  return seg_sum(P, edges_w)


def _tc_msg(h, Wm, bm):
  def body(h_ref, w_ref, b_ref, o_ref):
    o_ref[...] = jnp.maximum(
        jnp.dot(h_ref[...], w_ref[...], preferred_element_type=jnp.float32)
        + b_ref[...], 0.0)

  return pl.pallas_call(
      body,
      grid=(N // BM,),
      in_specs=[
          pl.BlockSpec((BM, D), lambda i: (i, 0)),
          pl.BlockSpec((D, D), lambda i: (0, 0)),
          pl.BlockSpec((1, D), lambda i: (0, 0)),
      ],
      out_specs=pl.BlockSpec((BM, D), lambda i: (i, 0)),
      out_shape=jax.ShapeDtypeStruct((N, D), jnp.float32),
  )(h, Wm, bm.reshape(1, D))


def _update_block(hv, a0, a1, wt, wb, b, gg, be):
  t = (jnp.dot(hv, wt, preferred_element_type=jnp.float32)
       + jnp.dot(a0 + a1, wb, preferred_element_type=jnp.float32) + b)
  hn = jnp.maximum(hv + jnp.maximum(t, 0.0), 0.0)
  mu = jnp.mean(hn, axis=1, keepdims=True)
  var = jnp.mean((hn - mu) ** 2, axis=1, keepdims=True)
  return (hn - mu) * lax.rsqrt(var + EPS) * gg + be


_AGG_SPECS = [
    pl.BlockSpec((BM, D), lambda i: (i, 0)),
    pl.BlockSpec((1, BM, D), lambda i: (0, i, 0)),
    pl.BlockSpec((1, BM, D), lambda i: (1, i, 0)),
    pl.BlockSpec((D, D), lambda i: (0, 0)),
    pl.BlockSpec((D, D), lambda i: (1, 0)),
    pl.BlockSpec((1, D), lambda i: (0, 0)),
    pl.BlockSpec((1, D), lambda i: (0, 0)),
    pl.BlockSpec((1, D), lambda i: (0, 0)),
]


def _tc_update_msg(h, agg2, Wn, bn, g, be, Wm_next, bm_next):
  def body(h_ref, a0_ref, a1_ref, wt_ref, wb_ref, b_ref, g_ref, be_ref,
           wm_ref, bm_ref, o_ref, p_ref):
    y = _update_block(h_ref[...], a0_ref[0], a1_ref[0], wt_ref[...],
                      wb_ref[...], b_ref[...], g_ref[...], be_ref[...])
    o_ref[...] = y
    p_ref[...] = jnp.maximum(
        jnp.dot(y, wm_ref[...], preferred_element_type=jnp.float32)
        + bm_ref[...], 0.0)

  return pl.pallas_call(
      body,
      grid=(N // BM,),
      in_specs=_AGG_SPECS + [
          pl.BlockSpec((D, D), lambda i: (0, 0)),
          pl.BlockSpec((1, D), lambda i: (0, 0)),
      ],
      out_specs=[
          pl.BlockSpec((BM, D), lambda i: (i, 0)),
          pl.BlockSpec((BM, D), lambda i: (i, 0)),
      ],
      out_shape=[
          jax.ShapeDtypeStruct((N, D), jnp.float32),
          jax.ShapeDtypeStruct((N, D), jnp.float32),
      ],
  )(h, agg2, agg2, Wn, Wn, bn.reshape(1, D), g.reshape(1, D),
    be.reshape(1, D), Wm_next, bm_next.reshape(1, D))


def _tc_update_final(h, agg2, Wn, bn, g, be):
  def body(h_ref, a0_ref, a1_ref, wt_ref, wb_ref, b_ref, g_ref, be_ref,
           o_ref, ctx_ref):
    y = _update_block(h_ref[...], a0_ref[0], a1_ref[0], wt_ref[...],
                      wb_ref[...], b_ref[...], g_ref[...], be_ref[...])
    o_ref[...] = y

    @pl.when(pl.program_id(0) == 0)
    def _init():
      ctx_ref[...] = jnp.zeros_like(ctx_ref)

    ctx_ref[...] += jnp.sum(y, axis=0, keepdims=True)

    @pl.when(pl.program_id(0) == N // BM - 1)
    def _fin():
      ctx_ref[...] = ctx_ref[...] * (1.0 / N)

  return pl.pallas_call(
      body,
      grid=(N // BM,),
      in_specs=_AGG_SPECS,
      out_specs=[
          pl.BlockSpec((BM, D), lambda i: (i, 0)),
          pl.BlockSpec((1, D), lambda i: (0, 0)),
      ],
      out_shape=[
          jax.ShapeDtypeStruct((N, D), jnp.float32),
          jax.ShapeDtypeStruct((1, D), jnp.float32),
      ],
  )(h, agg2, agg2, Wn, Wn, bn.reshape(1, D), g.reshape(1, D),
    be.reshape(1, D))


def kernel(x, edge_index, Wm0, bm0, Wn0, bn0, g0, be0, Wm1, bm1, Wn1, bn1,
           g1, be1):
  edges_w = edge_index.reshape(2 * E)

  P0 = _tc_msg(x, Wm0, bm0)
  agg0 = _sc_segment_sum(P0, edges_w)
  h1, P1 = _tc_update_msg(x, agg0, Wn0, bn0, g0, be0, Wm1, bm1)
  agg1 = _sc_segment_sum(P1, edges_w)
  h2, ctx = _tc_update_final(h1, agg1, Wn1, bn1, g1, be1)
  return (h2, ctx)
